# baseline (speedup 1.0000x reference)
import functools

import jax
import jax.numpy as jnp
from jax.experimental import pallas as pl
from jax.experimental.pallas import tpu as pltpu

_TTI = 128
_TTJ = 64
_TMI = 128
_TMJ = 64
_G = 8


def _bdiag(w, g=_G):
    k, nn = w.shape
    out = jnp.zeros((g * k, g * nn), jnp.float32)
    for q in range(g):
        out = out.at[q * k:(q + 1) * k, q * nn:(q + 1) * nn].set(w)
    return out


def _tile(w, g=_G):
    return jnp.concatenate([w] * g, axis=-1)


def _rep16(w):
    return jnp.broadcast_to(w, (w.shape[0], 16))


def _pack_mlp(p):
    l, a = p["logic"], p["alpha"]
    wc = jnp.concatenate([l["W1"], a["W1"]], axis=1)
    bc = jnp.concatenate([l["b1"], a["b1"]])[None, :]
    w2 = jnp.zeros((64, 17), jnp.float32)
    w2 = w2.at[0:32, 0:16].set(l["W2"]).at[32:64, 16:17].set(a["W2"])
    b2 = jnp.concatenate([l["b2"], a["b2"]])[None, :]
    return wc, bc, w2, b2


def _pack_l02(p):
    l, a = p["logic"], p["alpha"]
    perm = lambda w: jnp.concatenate([w[32:64], w[0:32]], axis=0)
    wc = jnp.concatenate([l["W1"], a["W1"], perm(l["W1"]), perm(a["W1"])], axis=1)
    bc = jnp.concatenate([l["b1"], a["b1"], l["b1"], a["b1"]])[None, :]
    w_xi, w_a, w_xj, w_b = wc[0:16], wc[16:32], wc[32:48], wc[48:64]
    w2l = jnp.zeros((128, 32), jnp.float32)
    w2l = w2l.at[0:32, 0:16].set(l["W2"]).at[64:96, 16:32].set(l["W2"])
    w2a = jnp.zeros((128, 32), jnp.float32)
    w2a = w2a.at[32:64, 0:16].set(_rep16(a["W2"]))
    w2a = w2a.at[96:128, 16:32].set(_rep16(a["W2"]))
    b2l = jnp.concatenate([l["b2"], l["b2"]])[None, :]
    b2a = _tile(_rep16(a["b2"][None]), 2)
    wap = _bdiag(w_a)
    wbp = _bdiag(w_b)
    wxit = _tile(w_xi)
    bct = _tile(bc)
    w2p = jnp.concatenate([_bdiag(0.5 * w2l), _bdiag(0.5 * w2a)], axis=1)
    b2p = jnp.concatenate([_tile(0.5 * b2l), _tile(0.5 * b2a)], axis=1)
    return wap, wbp, wxit, bct, _bdiag(w_xj), w2p, b2p


def _pack_l12(p):
    l, a = p["logic"], p["alpha"]
    wc = jnp.concatenate([l["W1"], a["W1"]], axis=1)
    bc = jnp.concatenate([l["b1"], a["b1"]])[None, :]
    w_ui, w_t, w_uj, w_tp = wc[0:16], wc[16:32], wc[32:48], wc[48:64]
    wq = 0.25 * jnp.concatenate([w_t, w_tp], axis=0)
    wqp = _bdiag(wq)
    wuit = _tile(w_ui)
    bdt = _tile(bc)
    w2l1 = jnp.zeros((64, 16), jnp.float32).at[0:32].set(l["W2"])
    w2a1 = jnp.zeros((64, 16), jnp.float32).at[32:64].set(_rep16(a["W2"]))
    w2p = jnp.concatenate([_bdiag(0.5 * w2l1), _bdiag(0.5 * w2a1)], axis=1)
    b2p = jnp.concatenate([_tile(0.5 * l["b2"][None]),
                           _tile(0.5 * _rep16(a["b2"][None]))], axis=1)
    return wqp, wuit, bdt, _bdiag(w_uj), w2p, b2p


def _dot(x, w):
    return jnp.dot(x, w, preferred_element_type=jnp.float32)


def _sig(x):
    return 0.5 * jnp.tanh(0.5 * x) + 0.5


def _mlp2(x, wc, bc, w2, b2):
    h = jnp.maximum(_dot(x, wc) + bc, 0.0)
    g = _dot(h, w2) + b2
    return _sig(g[:, 0:16]) * _sig(g[:, 16:17])


def _kernel_t(x2_ref, x1_ref, wc0, bc0, w20, b20, wc1, bc1, w21, b21,
              x2t_ref, out00_ref, out10_ref, red_ref, *, nio):
    b = pl.program_id(0)
    io = pl.program_id(1)
    jo = pl.program_id(2)
    blk = x2_ref[0]
    ttj, tti, _ = blk.shape
    x2t_ref[0] = jnp.swapaxes(blk, 0, 1)

    rr = jax.lax.broadcasted_iota(jnp.int32, (ttj, tti, 1), 0) + jo * ttj
    cc = jax.lax.broadcasted_iota(jnp.int32, (ttj, tti, 1), 1) + io * tti
    eq = rr == cc
    ex = jnp.max(jnp.where(eq, 0.0, blk), axis=1)
    fa = jnp.min(jnp.where(eq, 1.0, blk), axis=1)
    prev = red_ref[pl.ds(jo * ttj, ttj), :]
    ex = jnp.where(io == 0, ex, jnp.maximum(prev[:, 0:16], ex))
    fa = jnp.where(io == 0, fa, jnp.minimum(prev[:, 16:32], fa))
    red_ref[pl.ds(jo * ttj, ttj), :] = jnp.concatenate([ex, fa], axis=-1)

    @pl.when(io == nio - 1)
    def _():
        x1r = x1_ref[0, pl.ds(jo * ttj, ttj), :]
        red = red_ref[pl.ds(jo * ttj, ttj), :]
        f1 = jnp.concatenate([x1r, red], axis=-1)
        out10_ref[0] = _mlp2(f1, wc1[...], bc1[...], w21[...], b21[...])

    @pl.when(jnp.logical_and(io == 0, jo == 0))
    def _():
        x1f = x1_ref[0]
        r1 = jnp.concatenate([jnp.max(x1f, axis=0), jnp.min(x1f, axis=0)])[None, :]
        out00_ref[pl.ds(b, 1), :] = _mlp2(r1, wc0[...], bc0[...], w20[...], b20[...])


def _kernel_m(x2a_ref, x2b_ref, x1i_ref, x1jp_ref, u10_ref, u10p_ref, u00_ref,
              wap, wbp, wxit, bct, wxj, w2p, b2p,
              wqp, wuit, bdt, wuj, w2p1, b2p1,
              wd1, bd1, w2d1, b2d1,
              wd0, bd0, w2d0, b2d0,
              out2_ref, out1_ref, out0_ref, red_ref, *, nti, ntj):
    b = pl.program_id(0)
    i = pl.program_id(1)
    j = pl.program_id(2)
    ti = x1i_ref.shape[1]
    gb = x1jp_ref.shape[1]
    tj = gb * _G
    m8 = ti * gb

    a2 = x2a_ref[0].reshape(m8, 128)
    bt2 = x2b_ref[0].reshape(m8, 128)
    x1i = x1i_ref[0]
    x1jp = x1jp_ref[0]

    h2 = _dot(a2, wap[...]) + _dot(bt2, wbp[...])
    hxi = _dot(x1i, wxit[...]) + bct[...]
    hxj = _dot(x1jp, wxj[...])
    h3 = h2.reshape(ti, gb, 1024) + hxi[:, None, :] + hxj[None, :, :]
    h = jnp.maximum(h3, 0.0).reshape(m8, 1024)
    g = jnp.tanh(_dot(h, w2p[...]) + b2p[...])
    r = (1.0 + g[:, 0:256]) * (1.0 + g[:, 256:512])

    r3 = r.reshape(ti, gb, 256)
    ii = jax.lax.broadcasted_iota(jnp.int32, (ti, gb, 256), 0) + i * ti
    jb = jax.lax.broadcasted_iota(jnp.int32, (ti, gb, 256), 1)
    ln = jax.lax.broadcasted_iota(jnp.int32, (ti, gb, 256), 2)
    jj = j * tj + jb * _G + ln // 32
    eq = jnp.logical_and(ii == jj, (ln % 32) < 16)
    ex3 = jnp.where(eq, 0.0, r3)
    fa3 = jnp.where(eq, 4.0, r3)
    k = gb
    while k > 1:
        h_ = k // 2
        ex3 = jnp.maximum(ex3[:, :h_], ex3[:, h_:])
        fa3 = jnp.minimum(fa3[:, :h_], fa3[:, h_:])
        k = h_
    ex2 = ex3[:, 0]
    fa2 = fa3[:, 0]
    w_ = 256
    while w_ > 32:
        h_ = w_ // 2
        ex2 = jnp.maximum(ex2[:, :h_], ex2[:, h_:])
        fa2 = jnp.minimum(fa2[:, :h_], fa2[:, h_:])
        w_ = h_
    ex = ex2[:, 0:16]
    fa = fa2[:, 0:16]
    prev = red_ref[...]
    ex = jnp.where(j == 0, ex, jnp.maximum(prev[:, 0:16], ex))
    fa = jnp.where(j == 0, fa, jnp.minimum(prev[:, 16:32], fa))
    red_ref[...] = jnp.concatenate([ex, fa], axis=-1)

    u1i = u10_ref[0, pl.ds(i * ti, ti), :]
    u1jp = u10p_ref[0, pl.ds(j * gb, gb), :]
    q2 = _dot(r, wqp[...])
    qxi = _dot(u1i, wuit[...]) + bdt[...]
    qxj = _dot(u1jp, wuj[...])
    q3 = q2.reshape(ti, gb, 512) + qxi[:, None, :] + qxj[None, :, :]
    h1 = jnp.maximum(q3, 0.0).reshape(m8, 512)
    g1 = jnp.tanh(_dot(h1, w2p1[...]) + b2p1[...])
    o = (0.25 + 0.25 * g1[:, 0:128]) * (1.0 + g1[:, 128:256])
    out2_ref[0] = o.reshape(ti, gb, 128)

    @pl.when(j == ntj - 1)
    def _():
        red = red_ref[...]
        u00b = jnp.broadcast_to(u00_ref[pl.ds(b, 1), :], (ti, 16))
        f1 = jnp.concatenate([u00b, u1i, red], axis=-1)
        out1_ref[0] = _mlp2(f1, wd1[...], bd1[...], w2d1[...], b2d1[...])

    @pl.when(jnp.logical_and(j == ntj - 1, i == nti - 1))
    def _():
        u1f = u10_ref[0]
        r1 = jnp.concatenate([jnp.max(u1f, axis=0), jnp.min(u1f, axis=0)])[None, :]
        f0 = jnp.concatenate([u00_ref[pl.ds(b, 1), :], r1], axis=-1)
        out0_ref[pl.ds(b, 1), :] = _mlp2(f0, wd0[...], bd0[...], w2d0[...], b2d0[...])


@jax.jit
def kernel(x1, x2, params):
    bsz, n, c = x1.shape

    p00, p01, p02 = params[0]
    p10, p11, p12 = params[1]
    wa0 = _pack_mlp(p00)
    wa1 = _pack_mlp(p01)
    wm2 = _pack_l02(p02)
    wq2 = _pack_l12(p12)
    wd1 = _pack_mlp(p11)
    wd1 = (wd1[0].at[32:64].multiply(0.25), *wd1[1:])
    wd0 = _pack_mlp(p10)

    tti, ttj = _TTI, _TTJ
    nio, njo = n // tti, n // ttj
    wfull_t = [pl.BlockSpec(w.shape, functools.partial(lambda nd, b, i, j: (0,) * nd, w.ndim))
               for w in (*wa0, *wa1)]
    x2t, out00, out10 = pl.pallas_call(
        functools.partial(_kernel_t, nio=nio),
        grid=(bsz, nio, njo),
        in_specs=[
            pl.BlockSpec((1, ttj, tti, c), lambda b, io, jo: (b, jo, io, 0)),
            pl.BlockSpec((1, n, c), lambda b, io, jo: (b, 0, 0)),
            *wfull_t,
        ],
        out_specs=[
            pl.BlockSpec((1, tti, ttj, c), lambda b, io, jo: (b, io, jo, 0)),
            pl.BlockSpec((bsz, c), lambda b, io, jo: (0, 0)),
            pl.BlockSpec((1, ttj, c), lambda b, io, jo: (b, jo, 0)),
        ],
        out_shape=[
            jax.ShapeDtypeStruct((bsz, n, n, c), jnp.float32),
            jax.ShapeDtypeStruct((bsz, c), jnp.float32),
            jax.ShapeDtypeStruct((bsz, n, c), jnp.float32),
        ],
        scratch_shapes=[pltpu.VMEM((n, 2 * c), jnp.float32)],
    )(x2, x1, *wa0, *wa1)

    ti, tj = _TMI, _TMJ
    nti, ntj = n // ti, n // tj
    x2p = x2.reshape(bsz, n, n // _G, _G * c)
    x2tp = x2t.reshape(bsz, n, n // _G, _G * c)
    weights_m = (*wm2, *wq2, *wd1, *wd0)
    wfull_m = [pl.BlockSpec(w.shape, functools.partial(lambda nd, b, i, j: (0,) * nd, w.ndim))
               for w in weights_m]
    gb = tj // _G
    out2p, out1, out0 = pl.pallas_call(
        functools.partial(_kernel_m, nti=nti, ntj=ntj),
        grid=(bsz, nti, ntj),
        in_specs=[
            pl.BlockSpec((1, ti, gb, _G * c), lambda b, i, j: (b, i, j, 0)),
            pl.BlockSpec((1, ti, gb, _G * c), lambda b, i, j: (b, i, j, 0)),
            pl.BlockSpec((1, ti, c), lambda b, i, j: (b, i, 0)),
            pl.BlockSpec((1, gb, _G * c), lambda b, i, j: (b, j, 0)),
            pl.BlockSpec((1, n, c), lambda b, i, j: (b, 0, 0)),
            pl.BlockSpec((1, n // _G, _G * c), lambda b, i, j: (b, 0, 0)),
            pl.BlockSpec((bsz, c), lambda b, i, j: (0, 0)),
            *wfull_m,
        ],
        out_specs=[
            pl.BlockSpec((1, ti, gb, _G * c), lambda b, i, j: (b, i, j, 0)),
            pl.BlockSpec((1, ti, c), lambda b, i, j: (b, i, 0)),
            pl.BlockSpec((bsz, c), lambda b, i, j: (0, 0)),
        ],
        out_shape=[
            jax.ShapeDtypeStruct((bsz, n, n // _G, _G * c), jnp.float32),
            jax.ShapeDtypeStruct((bsz, n, c), jnp.float32),
            jax.ShapeDtypeStruct((bsz, c), jnp.float32),
        ],
        scratch_shapes=[pltpu.VMEM((ti, 2 * c), jnp.float32)],
    )(x2p, x2tp, x1, x1.reshape(bsz, n // _G, _G * c), out10,
      out10.reshape(bsz, n // _G, _G * c), out00, *weights_m)

    return (out0, out1, out2p.reshape(bsz, n, n, c))


# speedup vs baseline: 1.7621x; 1.7621x over previous
import functools

import jax
import jax.numpy as jnp
from jax.experimental import pallas as pl
from jax.experimental.pallas import tpu as pltpu

_TTI = 128
_TTJ = 64
_TMI = 128
_TMJ = 64
_G = 8


def _bdiag(w, g=_G):
    k, nn = w.shape
    out = jnp.zeros((g * k, g * nn), jnp.float32)
    for q in range(g):
        out = out.at[q * k:(q + 1) * k, q * nn:(q + 1) * nn].set(w)
    return out


def _tile(w, g=_G):
    return jnp.concatenate([w] * g, axis=-1)


def _rep16(w):
    return jnp.broadcast_to(w, (w.shape[0], 16))


def _pack_mlp(p):
    l, a = p["logic"], p["alpha"]
    wc = jnp.concatenate([l["W1"], a["W1"]], axis=1)
    bc = jnp.concatenate([l["b1"], a["b1"]])[None, :]
    w2 = jnp.zeros((64, 17), jnp.float32)
    w2 = w2.at[0:32, 0:16].set(l["W2"]).at[32:64, 16:17].set(a["W2"])
    b2 = jnp.concatenate([l["b2"], a["b2"]])[None, :]
    return wc, bc, w2, b2


def _pack_l02(p):
    l, a = p["logic"], p["alpha"]
    perm = lambda w: jnp.concatenate([w[32:64], w[0:32]], axis=0)
    wc = jnp.concatenate([l["W1"], a["W1"], perm(l["W1"]), perm(a["W1"])], axis=1)
    bc = jnp.concatenate([l["b1"], a["b1"], l["b1"], a["b1"]])[None, :]
    w_xi, w_a, w_xj, w_b = wc[0:16], wc[16:32], wc[32:48], wc[48:64]
    w2l = jnp.zeros((128, 32), jnp.float32)
    w2l = w2l.at[0:32, 0:16].set(l["W2"]).at[64:96, 16:32].set(l["W2"])
    w2a = jnp.zeros((128, 32), jnp.float32)
    w2a = w2a.at[32:64, 0:16].set(_rep16(a["W2"]))
    w2a = w2a.at[96:128, 16:32].set(_rep16(a["W2"]))
    b2l = jnp.concatenate([l["b2"], l["b2"]])[None, :]
    b2a = _tile(_rep16(a["b2"][None]), 2)
    wap = _bdiag(w_a)
    wbp = _bdiag(w_b)
    wxit = _tile(w_xi)
    bct = _tile(bc)
    w2p = jnp.concatenate([_bdiag(0.5 * w2l), _bdiag(0.5 * w2a)], axis=1)
    b2p = jnp.concatenate([_tile(0.5 * b2l), _tile(0.5 * b2a)], axis=1)
    return wap, wbp, wxit, bct, _bdiag(w_xj), w2p, b2p


def _pack_l12(p):
    l, a = p["logic"], p["alpha"]
    wc = jnp.concatenate([l["W1"], a["W1"]], axis=1)
    bc = jnp.concatenate([l["b1"], a["b1"]])[None, :]
    w_ui, w_t, w_uj, w_tp = wc[0:16], wc[16:32], wc[32:48], wc[48:64]
    wq = 0.25 * jnp.concatenate([w_t, w_tp], axis=0)
    wqp = _bdiag(wq)
    wuit = _tile(w_ui)
    bdt = _tile(bc)
    w2l1 = jnp.zeros((64, 16), jnp.float32).at[0:32].set(l["W2"])
    w2a1 = jnp.zeros((64, 16), jnp.float32).at[32:64].set(_rep16(a["W2"]))
    w2p = jnp.concatenate([_bdiag(0.5 * w2l1), _bdiag(0.5 * w2a1)], axis=1)
    b2p = jnp.concatenate([_tile(0.5 * l["b2"][None]),
                           _tile(0.5 * _rep16(a["b2"][None]))], axis=1)
    return wqp, wuit, bdt, _bdiag(w_uj), w2p, b2p


def _dot(x, w):
    return jnp.dot(x, w, preferred_element_type=jnp.float32)


def _sig(x):
    return 0.5 * jnp.tanh(0.5 * x) + 0.5


def _mlp2(x, wc, bc, w2, b2):
    h = jnp.maximum(_dot(x, wc) + bc, 0.0)
    g = _dot(h, w2) + b2
    return _sig(g[:, 0:16]) * _sig(g[:, 16:17])


def _kernel_a(x2_ref, x1_ref, wc0, bc0, w20, b20, wc1, bc1, w21, b21,
              out00_ref, out10_ref, red_ref, *, nt):
    b = pl.program_id(0)
    i = pl.program_id(1)
    j = pl.program_id(2)
    t = out10_ref.shape[1]
    blk = x2_ref[0]
    w = blk.shape[1]
    ii = jax.lax.broadcasted_iota(jnp.int32, (t, w), 0) + i * t
    jl = jax.lax.broadcasted_iota(jnp.int32, (t, w), 1) // 16 + j * t
    eq = ii == jl
    ex2 = jnp.where(eq, 0.0, blk)
    fa2 = jnp.where(eq, 1.0, blk)
    k = w // 16
    while k > 8:
        h = (k // 2) * 16
        ex2 = jnp.maximum(ex2[:, :h], ex2[:, h:])
        fa2 = jnp.minimum(fa2[:, :h], fa2[:, h:])
        k = k // 2
    ex3 = ex2.reshape(t, k, 16)
    fa3 = fa2.reshape(t, k, 16)
    ex = jnp.max(ex3, axis=1)
    fa = jnp.min(fa3, axis=1)
    prev = red_ref[...]
    ex = jnp.where(j == 0, ex, jnp.maximum(prev[:, 0:16], ex))
    fa = jnp.where(j == 0, fa, jnp.minimum(prev[:, 16:32], fa))
    red_ref[...] = jnp.concatenate([ex, fa], axis=-1)

    @pl.when(j == nt - 1)
    def _():
        x1i = x1_ref[0, pl.ds(i * t, t), :]
        red = red_ref[...]
        f1 = jnp.concatenate([x1i, red], axis=-1)
        out10_ref[0] = _mlp2(f1, wc1[...], bc1[...], w21[...], b21[...])

    @pl.when(jnp.logical_and(i == 0, j == 0))
    def _():
        x1f = x1_ref[0]
        r1 = jnp.concatenate([jnp.max(x1f, axis=0), jnp.min(x1f, axis=0)])[None, :]
        out00_ref[pl.ds(b, 1), :] = _mlp2(r1, wc0[...], bc0[...], w20[...], b20[...])


def _kernel_m(x2a_ref, x2b_ref, x1i_ref, x1jp_ref, u10_ref, u10p_ref, u00_ref,
              wap, wbp, wxit, bct, wxj, w2p, b2p,
              wqp, wuit, bdt, wuj, w2p1, b2p1,
              wd1, bd1, w2d1, b2d1,
              wd0, bd0, w2d0, b2d0,
              out2_ref, out1_ref, out0_ref, red_ref, *, nti, ntj):
    b = pl.program_id(0)
    i = pl.program_id(1)
    j = pl.program_id(2)
    ti = x1i_ref.shape[1]
    gb = x1jp_ref.shape[1]
    tj = gb * _G
    m8 = ti * gb

    a2 = x2a_ref[0].reshape(m8, 128)
    bt2 = x2b_ref[0].reshape(m8, 128)
    x1i = x1i_ref[0]
    x1jp = x1jp_ref[0]

    h2 = _dot(a2, wap[...]) + _dot(bt2, wbp[...])
    hxi = _dot(x1i, wxit[...]) + bct[...]
    hxj = _dot(x1jp, wxj[...])
    h3 = h2.reshape(ti, gb, 1024) + hxi[:, None, :] + hxj[None, :, :]
    h = jnp.maximum(h3, 0.0).reshape(m8, 1024)
    g = jnp.tanh(_dot(h, w2p[...]) + b2p[...])
    r = (1.0 + g[:, 0:256]) * (1.0 + g[:, 256:512])

    r3 = r.reshape(ti, gb, 256)
    ii = jax.lax.broadcasted_iota(jnp.int32, (ti, gb, 256), 0) + i * ti
    jb = jax.lax.broadcasted_iota(jnp.int32, (ti, gb, 256), 1)
    ln = jax.lax.broadcasted_iota(jnp.int32, (ti, gb, 256), 2)
    jj = j * tj + jb * _G + ln // 32
    eq = jnp.logical_and(ii == jj, (ln % 32) < 16)
    ex3 = jnp.where(eq, 0.0, r3)
    fa3 = jnp.where(eq, 4.0, r3)
    k = gb
    while k > 1:
        h_ = k // 2
        ex3 = jnp.maximum(ex3[:, :h_], ex3[:, h_:])
        fa3 = jnp.minimum(fa3[:, :h_], fa3[:, h_:])
        k = h_
    ex2 = ex3[:, 0]
    fa2 = fa3[:, 0]
    w_ = 256
    while w_ > 32:
        h_ = w_ // 2
        ex2 = jnp.maximum(ex2[:, :h_], ex2[:, h_:])
        fa2 = jnp.minimum(fa2[:, :h_], fa2[:, h_:])
        w_ = h_
    ex = ex2[:, 0:16]
    fa = fa2[:, 0:16]
    prev = red_ref[...]
    ex = jnp.where(j == 0, ex, jnp.maximum(prev[:, 0:16], ex))
    fa = jnp.where(j == 0, fa, jnp.minimum(prev[:, 16:32], fa))
    red_ref[...] = jnp.concatenate([ex, fa], axis=-1)

    u1i = u10_ref[0, pl.ds(i * ti, ti), :]
    u1jp = u10p_ref[0, pl.ds(j * gb, gb), :]
    q2 = _dot(r, wqp[...])
    qxi = _dot(u1i, wuit[...]) + bdt[...]
    qxj = _dot(u1jp, wuj[...])
    q3 = q2.reshape(ti, gb, 512) + qxi[:, None, :] + qxj[None, :, :]
    h1 = jnp.maximum(q3, 0.0).reshape(m8, 512)
    g1 = jnp.tanh(_dot(h1, w2p1[...]) + b2p1[...])
    o = (0.25 + 0.25 * g1[:, 0:128]) * (1.0 + g1[:, 128:256])
    out2_ref[0] = o.reshape(ti, gb, 128)

    @pl.when(j == ntj - 1)
    def _():
        red = red_ref[...]
        u00b = jnp.broadcast_to(u00_ref[pl.ds(b, 1), :], (ti, 16))
        f1 = jnp.concatenate([u00b, u1i, red], axis=-1)
        out1_ref[0] = _mlp2(f1, wd1[...], bd1[...], w2d1[...], b2d1[...])

    @pl.when(jnp.logical_and(j == ntj - 1, i == nti - 1))
    def _():
        u1f = u10_ref[0]
        r1 = jnp.concatenate([jnp.max(u1f, axis=0), jnp.min(u1f, axis=0)])[None, :]
        f0 = jnp.concatenate([u00_ref[pl.ds(b, 1), :], r1], axis=-1)
        out0_ref[pl.ds(b, 1), :] = _mlp2(f0, wd0[...], bd0[...], w2d0[...], b2d0[...])


@jax.jit
def kernel(x1, x2, params):
    bsz, n, c = x1.shape

    p00, p01, p02 = params[0]
    p10, p11, p12 = params[1]
    wa0 = _pack_mlp(p00)
    wa1 = _pack_mlp(p01)
    wm2 = _pack_l02(p02)
    wq2 = _pack_l12(p12)
    wd1 = _pack_mlp(p11)
    wd1 = (wd1[0].at[32:64].multiply(0.25), *wd1[1:])
    wd0 = _pack_mlp(p10)

    x2t = jnp.swapaxes(x2, 1, 2)
    x2r = x2.reshape(bsz, n, n * c)
    ta = _TTI
    nta = n // ta
    wfull_t = [pl.BlockSpec(w.shape, functools.partial(lambda nd, b, i, j: (0,) * nd, w.ndim))
               for w in (*wa0, *wa1)]
    out00, out10 = pl.pallas_call(
        functools.partial(_kernel_a, nt=nta),
        grid=(bsz, nta, nta),
        in_specs=[
            pl.BlockSpec((1, ta, ta * c), lambda b, i, j: (b, i, j)),
            pl.BlockSpec((1, n, c), lambda b, i, j: (b, 0, 0)),
            *wfull_t,
        ],
        out_specs=[
            pl.BlockSpec((bsz, c), lambda b, i, j: (0, 0)),
            pl.BlockSpec((1, ta, c), lambda b, i, j: (b, i, 0)),
        ],
        out_shape=[
            jax.ShapeDtypeStruct((bsz, c), jnp.float32),
            jax.ShapeDtypeStruct((bsz, n, c), jnp.float32),
        ],
        scratch_shapes=[pltpu.VMEM((ta, 2 * c), jnp.float32)],
    )(x2r, x1, *wa0, *wa1)

    ti, tj = _TMI, _TMJ
    nti, ntj = n // ti, n // tj
    x2p = x2.reshape(bsz, n, n // _G, _G * c)
    x2tp = x2t.reshape(bsz, n, n // _G, _G * c)
    weights_m = (*wm2, *wq2, *wd1, *wd0)
    wfull_m = [pl.BlockSpec(w.shape, functools.partial(lambda nd, b, i, j: (0,) * nd, w.ndim))
               for w in weights_m]
    gb = tj // _G
    out2p, out1, out0 = pl.pallas_call(
        functools.partial(_kernel_m, nti=nti, ntj=ntj),
        grid=(bsz, nti, ntj),
        in_specs=[
            pl.BlockSpec((1, ti, gb, _G * c), lambda b, i, j: (b, i, j, 0)),
            pl.BlockSpec((1, ti, gb, _G * c), lambda b, i, j: (b, i, j, 0)),
            pl.BlockSpec((1, ti, c), lambda b, i, j: (b, i, 0)),
            pl.BlockSpec((1, gb, _G * c), lambda b, i, j: (b, j, 0)),
            pl.BlockSpec((1, n, c), lambda b, i, j: (b, 0, 0)),
            pl.BlockSpec((1, n // _G, _G * c), lambda b, i, j: (b, 0, 0)),
            pl.BlockSpec((bsz, c), lambda b, i, j: (0, 0)),
            *wfull_m,
        ],
        out_specs=[
            pl.BlockSpec((1, ti, gb, _G * c), lambda b, i, j: (b, i, j, 0)),
            pl.BlockSpec((1, ti, c), lambda b, i, j: (b, i, 0)),
            pl.BlockSpec((bsz, c), lambda b, i, j: (0, 0)),
        ],
        out_shape=[
            jax.ShapeDtypeStruct((bsz, n, n // _G, _G * c), jnp.float32),
            jax.ShapeDtypeStruct((bsz, n, c), jnp.float32),
            jax.ShapeDtypeStruct((bsz, c), jnp.float32),
        ],
        scratch_shapes=[pltpu.VMEM((ti, 2 * c), jnp.float32)],
    )(x2p, x2tp, x1, x1.reshape(bsz, n // _G, _G * c), out10,
      out10.reshape(bsz, n // _G, _G * c), out00, *weights_m)

    return (out0, out1, out2p.reshape(bsz, n, n, c))


# speedup vs baseline: 1.8237x; 1.0350x over previous
import functools

import jax
import jax.numpy as jnp
from jax.experimental import pallas as pl
from jax.experimental.pallas import tpu as pltpu

_TTI = 128
_TTJ = 64
_TMI = 128
_TMJ = 64
_G = 8


def _bdiag(w, g=_G):
    k, nn = w.shape
    out = jnp.zeros((g * k, g * nn), jnp.float32)
    for q in range(g):
        out = out.at[q * k:(q + 1) * k, q * nn:(q + 1) * nn].set(w)
    return out


def _tile(w, g=_G):
    return jnp.concatenate([w] * g, axis=-1)


def _rep16(w):
    return jnp.broadcast_to(w, (w.shape[0], 16))


def _pack_mlp(p):
    l, a = p["logic"], p["alpha"]
    wc = jnp.concatenate([l["W1"], a["W1"]], axis=1)
    bc = jnp.concatenate([l["b1"], a["b1"]])[None, :]
    w2 = jnp.zeros((64, 17), jnp.float32)
    w2 = w2.at[0:32, 0:16].set(l["W2"]).at[32:64, 16:17].set(a["W2"])
    b2 = jnp.concatenate([l["b2"], a["b2"]])[None, :]
    return wc, bc, w2, b2


def _pack_l02(p):
    l, a = p["logic"], p["alpha"]
    perm = lambda w: jnp.concatenate([w[32:64], w[0:32]], axis=0)
    wc = jnp.concatenate([l["W1"], a["W1"], perm(l["W1"]), perm(a["W1"])], axis=1)
    bc = jnp.concatenate([l["b1"], a["b1"], l["b1"], a["b1"]])[None, :]
    w_xi, w_a, w_xj, w_b = wc[0:16], wc[16:32], wc[32:48], wc[48:64]
    w2l = jnp.zeros((128, 32), jnp.float32)
    w2l = w2l.at[0:32, 0:16].set(l["W2"]).at[64:96, 16:32].set(l["W2"])
    w2a = jnp.zeros((128, 32), jnp.float32)
    w2a = w2a.at[32:64, 0:16].set(_rep16(a["W2"]))
    w2a = w2a.at[96:128, 16:32].set(_rep16(a["W2"]))
    b2l = jnp.concatenate([l["b2"], l["b2"]])[None, :]
    b2a = _tile(_rep16(a["b2"][None]), 2)
    wap = _bdiag(w_a)
    wbp = _bdiag(w_b)
    wxit = _tile(w_xi)
    bct = _tile(bc)
    w2p = jnp.concatenate([_bdiag(0.5 * w2l), _bdiag(0.5 * w2a)], axis=1)
    b2p = jnp.concatenate([_tile(0.5 * b2l), _tile(0.5 * b2a)], axis=1)
    return wap, wbp, wxit, bct, _bdiag(w_xj), w2p, b2p


def _pack_l12(p):
    l, a = p["logic"], p["alpha"]
    wc = jnp.concatenate([l["W1"], a["W1"]], axis=1)
    bc = jnp.concatenate([l["b1"], a["b1"]])[None, :]
    w_ui, w_t, w_uj, w_tp = wc[0:16], wc[16:32], wc[32:48], wc[48:64]
    wq = 0.25 * jnp.concatenate([w_t, w_tp], axis=0)
    wqp = _bdiag(wq)
    wuit = _tile(w_ui)
    bdt = _tile(bc)
    w2l1 = jnp.zeros((64, 16), jnp.float32).at[0:32].set(l["W2"])
    w2a1 = jnp.zeros((64, 16), jnp.float32).at[32:64].set(_rep16(a["W2"]))
    w2p = jnp.concatenate([_bdiag(0.5 * w2l1), _bdiag(0.5 * w2a1)], axis=1)
    b2p = jnp.concatenate([_tile(0.5 * l["b2"][None]),
                           _tile(0.5 * _rep16(a["b2"][None]))], axis=1)
    return wqp, wuit, bdt, _bdiag(w_uj), w2p, b2p


def _dot(x, w):
    return jnp.dot(x, w, preferred_element_type=jnp.float32)


def _sig(x):
    return 0.5 * jnp.tanh(0.5 * x) + 0.5


def _mlp2(x, wc, bc, w2, b2):
    h = jnp.maximum(_dot(x, wc) + bc, 0.0)
    g = _dot(h, w2) + b2
    return _sig(g[:, 0:16]) * _sig(g[:, 16:17])


def _kernel_a(x2_ref, x1_ref, wc0, bc0, w20, b20, wc1, bc1, w21, b21,
              out00_ref, out10_ref, red_ref, *, nt):
    b = pl.program_id(0)
    i = pl.program_id(1)
    j = pl.program_id(2)
    t = out10_ref.shape[1]
    blk = x2_ref[0]
    gb = blk.shape[1]
    ii = jax.lax.broadcasted_iota(jnp.int32, (t, gb, 128), 0) + i * t
    jb = jax.lax.broadcasted_iota(jnp.int32, (t, gb, 128), 1)
    ln = jax.lax.broadcasted_iota(jnp.int32, (t, gb, 128), 2)
    jl = j * t + jb * _G + ln // 16
    eq = ii == jl
    ex3 = jnp.where(eq, 0.0, blk)
    fa3 = jnp.where(eq, 1.0, blk)
    k = gb
    while k > 1:
        h = k // 2
        ex3 = jnp.maximum(ex3[:, :h], ex3[:, h:])
        fa3 = jnp.minimum(fa3[:, :h], fa3[:, h:])
        k = h
    ex2 = ex3[:, 0]
    fa2 = fa3[:, 0]
    w_ = 128
    while w_ > 16:
        h = w_ // 2
        ex2 = jnp.maximum(ex2[:, :h], ex2[:, h:])
        fa2 = jnp.minimum(fa2[:, :h], fa2[:, h:])
        w_ = h
    ex = ex2
    fa = fa2
    prev = red_ref[...]
    ex = jnp.where(j == 0, ex, jnp.maximum(prev[:, 0:16], ex))
    fa = jnp.where(j == 0, fa, jnp.minimum(prev[:, 16:32], fa))
    red_ref[...] = jnp.concatenate([ex, fa], axis=-1)

    @pl.when(j == nt - 1)
    def _():
        x1i = x1_ref[0, pl.ds(i * t, t), :]
        red = red_ref[...]
        f1 = jnp.concatenate([x1i, red], axis=-1)
        out10_ref[0] = _mlp2(f1, wc1[...], bc1[...], w21[...], b21[...])

    @pl.when(jnp.logical_and(i == 0, j == 0))
    def _():
        x1f = x1_ref[0]
        r1 = jnp.concatenate([jnp.max(x1f, axis=0), jnp.min(x1f, axis=0)])[None, :]
        out00_ref[pl.ds(b, 1), :] = _mlp2(r1, wc0[...], bc0[...], w20[...], b20[...])


def _kernel_m(x2a_ref, x2b_ref, x1i_ref, x1jp_ref, u10_ref, u10p_ref, u00_ref,
              wap, wbp, wxit, bct, wxj, w2p, b2p,
              wqp, wuit, bdt, wuj, w2p1, b2p1,
              wd1, bd1, w2d1, b2d1,
              wd0, bd0, w2d0, b2d0,
              out2_ref, out1_ref, out0_ref, red_ref, *, nti, ntj):
    b = pl.program_id(0)
    i = pl.program_id(1)
    j = pl.program_id(2)
    ti = x1i_ref.shape[1]
    gb = x1jp_ref.shape[1]
    tj = gb * _G
    m8 = ti * gb

    a2 = x2a_ref[0].reshape(m8, 128)
    bt2 = x2b_ref[0].reshape(m8, 128)
    x1i = x1i_ref[0]
    x1jp = x1jp_ref[0]

    h2 = _dot(a2, wap[...]) + _dot(bt2, wbp[...])
    hxi = _dot(x1i, wxit[...]) + bct[...]
    hxj = _dot(x1jp, wxj[...])
    h3 = h2.reshape(ti, gb, 1024) + hxi[:, None, :] + hxj[None, :, :]
    h = jnp.maximum(h3, 0.0).reshape(m8, 1024)
    g = jnp.tanh(_dot(h, w2p[...]) + b2p[...])
    r = (1.0 + g[:, 0:256]) * (1.0 + g[:, 256:512])

    r3 = r.reshape(ti, gb, 256)
    ii = jax.lax.broadcasted_iota(jnp.int32, (ti, gb, 256), 0) + i * ti
    jb = jax.lax.broadcasted_iota(jnp.int32, (ti, gb, 256), 1)
    ln = jax.lax.broadcasted_iota(jnp.int32, (ti, gb, 256), 2)
    jj = j * tj + jb * _G + ln // 32
    eq = jnp.logical_and(ii == jj, (ln % 32) < 16)
    ex3 = jnp.where(eq, 0.0, r3)
    fa3 = jnp.where(eq, 4.0, r3)
    k = gb
    while k > 1:
        h_ = k // 2
        ex3 = jnp.maximum(ex3[:, :h_], ex3[:, h_:])
        fa3 = jnp.minimum(fa3[:, :h_], fa3[:, h_:])
        k = h_
    ex2 = ex3[:, 0]
    fa2 = fa3[:, 0]
    w_ = 256
    while w_ > 32:
        h_ = w_ // 2
        ex2 = jnp.maximum(ex2[:, :h_], ex2[:, h_:])
        fa2 = jnp.minimum(fa2[:, :h_], fa2[:, h_:])
        w_ = h_
    ex = ex2[:, 0:16]
    fa = fa2[:, 0:16]
    prev = red_ref[...]
    ex = jnp.where(j == 0, ex, jnp.maximum(prev[:, 0:16], ex))
    fa = jnp.where(j == 0, fa, jnp.minimum(prev[:, 16:32], fa))
    red_ref[...] = jnp.concatenate([ex, fa], axis=-1)

    u1i = u10_ref[0, pl.ds(i * ti, ti), :]
    u1jp = u10p_ref[0, pl.ds(j * gb, gb), :]
    q2 = _dot(r, wqp[...])
    qxi = _dot(u1i, wuit[...]) + bdt[...]
    qxj = _dot(u1jp, wuj[...])
    q3 = q2.reshape(ti, gb, 512) + qxi[:, None, :] + qxj[None, :, :]
    h1 = jnp.maximum(q3, 0.0).reshape(m8, 512)
    g1 = jnp.tanh(_dot(h1, w2p1[...]) + b2p1[...])
    o = (0.25 + 0.25 * g1[:, 0:128]) * (1.0 + g1[:, 128:256])
    out2_ref[0] = o.reshape(ti, gb, 128)

    @pl.when(j == ntj - 1)
    def _():
        red = red_ref[...]
        u00b = jnp.broadcast_to(u00_ref[pl.ds(b, 1), :], (ti, 16))
        f1 = jnp.concatenate([u00b, u1i, red], axis=-1)
        out1_ref[0] = _mlp2(f1, wd1[...], bd1[...], w2d1[...], b2d1[...])

    @pl.when(jnp.logical_and(j == ntj - 1, i == nti - 1))
    def _():
        u1f = u10_ref[0]
        r1 = jnp.concatenate([jnp.max(u1f, axis=0), jnp.min(u1f, axis=0)])[None, :]
        f0 = jnp.concatenate([u00_ref[pl.ds(b, 1), :], r1], axis=-1)
        out0_ref[pl.ds(b, 1), :] = _mlp2(f0, wd0[...], bd0[...], w2d0[...], b2d0[...])


@jax.jit
def kernel(x1, x2, params):
    bsz, n, c = x1.shape

    p00, p01, p02 = params[0]
    p10, p11, p12 = params[1]
    wa0 = _pack_mlp(p00)
    wa1 = _pack_mlp(p01)
    wm2 = _pack_l02(p02)
    wq2 = _pack_l12(p12)
    wd1 = _pack_mlp(p11)
    wd1 = (wd1[0].at[32:64].multiply(0.25), *wd1[1:])
    wd0 = _pack_mlp(p10)

    x2p = x2.reshape(bsz, n, n // _G, _G * c)
    x2tp = jnp.swapaxes(x2, 1, 2).reshape(bsz, n, n // _G, _G * c)
    ta = _TTI
    nta = n // ta
    wfull_t = [pl.BlockSpec(w.shape, functools.partial(lambda nd, b, i, j: (0,) * nd, w.ndim))
               for w in (*wa0, *wa1)]
    out00, out10 = pl.pallas_call(
        functools.partial(_kernel_a, nt=nta),
        grid=(bsz, nta, nta),
        in_specs=[
            pl.BlockSpec((1, ta, ta // _G, _G * c), lambda b, i, j: (b, i, j, 0)),
            pl.BlockSpec((1, n, c), lambda b, i, j: (b, 0, 0)),
            *wfull_t,
        ],
        out_specs=[
            pl.BlockSpec((bsz, c), lambda b, i, j: (0, 0)),
            pl.BlockSpec((1, ta, c), lambda b, i, j: (b, i, 0)),
        ],
        out_shape=[
            jax.ShapeDtypeStruct((bsz, c), jnp.float32),
            jax.ShapeDtypeStruct((bsz, n, c), jnp.float32),
        ],
        scratch_shapes=[pltpu.VMEM((ta, 2 * c), jnp.float32)],
    )(x2p, x1, *wa0, *wa1)

    ti, tj = _TMI, _TMJ
    nti, ntj = n // ti, n // tj
    weights_m = (*wm2, *wq2, *wd1, *wd0)
    wfull_m = [pl.BlockSpec(w.shape, functools.partial(lambda nd, b, i, j: (0,) * nd, w.ndim))
               for w in weights_m]
    gb = tj // _G
    out2p, out1, out0 = pl.pallas_call(
        functools.partial(_kernel_m, nti=nti, ntj=ntj),
        grid=(bsz, nti, ntj),
        in_specs=[
            pl.BlockSpec((1, ti, gb, _G * c), lambda b, i, j: (b, i, j, 0)),
            pl.BlockSpec((1, ti, gb, _G * c), lambda b, i, j: (b, i, j, 0)),
            pl.BlockSpec((1, ti, c), lambda b, i, j: (b, i, 0)),
            pl.BlockSpec((1, gb, _G * c), lambda b, i, j: (b, j, 0)),
            pl.BlockSpec((1, n, c), lambda b, i, j: (b, 0, 0)),
            pl.BlockSpec((1, n // _G, _G * c), lambda b, i, j: (b, 0, 0)),
            pl.BlockSpec((bsz, c), lambda b, i, j: (0, 0)),
            *wfull_m,
        ],
        out_specs=[
            pl.BlockSpec((1, ti, gb, _G * c), lambda b, i, j: (b, i, j, 0)),
            pl.BlockSpec((1, ti, c), lambda b, i, j: (b, i, 0)),
            pl.BlockSpec((bsz, c), lambda b, i, j: (0, 0)),
        ],
        out_shape=[
            jax.ShapeDtypeStruct((bsz, n, n // _G, _G * c), jnp.float32),
            jax.ShapeDtypeStruct((bsz, n, c), jnp.float32),
            jax.ShapeDtypeStruct((bsz, c), jnp.float32),
        ],
        scratch_shapes=[pltpu.VMEM((ti, 2 * c), jnp.float32)],
    )(x2p, x2tp, x1, x1.reshape(bsz, n // _G, _G * c), out10,
      out10.reshape(bsz, n // _G, _G * c), out00, *weights_m)

    return (out0, out1, out2p.reshape(bsz, n, n, c))


# speedup vs baseline: 1.9360x; 1.0615x over previous
import functools

import jax
import jax.numpy as jnp
from jax.experimental import pallas as pl
from jax.experimental.pallas import tpu as pltpu

_TTI = 128
_TTJ = 64
_TMI = 128
_TMJ = 128
_G = 8


def _bdiag(w, g=_G):
    k, nn = w.shape
    out = jnp.zeros((g * k, g * nn), jnp.float32)
    for q in range(g):
        out = out.at[q * k:(q + 1) * k, q * nn:(q + 1) * nn].set(w)
    return out


def _tile(w, g=_G):
    return jnp.concatenate([w] * g, axis=-1)


def _rep16(w):
    return jnp.broadcast_to(w, (w.shape[0], 16))


def _pack_mlp(p):
    l, a = p["logic"], p["alpha"]
    wc = jnp.concatenate([l["W1"], a["W1"]], axis=1)
    bc = jnp.concatenate([l["b1"], a["b1"]])[None, :]
    w2 = jnp.zeros((64, 17), jnp.float32)
    w2 = w2.at[0:32, 0:16].set(l["W2"]).at[32:64, 16:17].set(a["W2"])
    b2 = jnp.concatenate([l["b2"], a["b2"]])[None, :]
    return wc, bc, w2, b2


def _pack_l02(p):
    l, a = p["logic"], p["alpha"]
    perm = lambda w: jnp.concatenate([w[32:64], w[0:32]], axis=0)
    wc = jnp.concatenate([l["W1"], a["W1"], perm(l["W1"]), perm(a["W1"])], axis=1)
    bc = jnp.concatenate([l["b1"], a["b1"], l["b1"], a["b1"]])[None, :]
    w_xi, w_a, w_xj, w_b = wc[0:16], wc[16:32], wc[32:48], wc[48:64]
    w2l = jnp.zeros((128, 32), jnp.float32)
    w2l = w2l.at[0:32, 0:16].set(l["W2"]).at[64:96, 16:32].set(l["W2"])
    w2a = jnp.zeros((128, 32), jnp.float32)
    w2a = w2a.at[32:64, 0:16].set(_rep16(a["W2"]))
    w2a = w2a.at[96:128, 16:32].set(_rep16(a["W2"]))
    b2l = jnp.concatenate([l["b2"], l["b2"]])[None, :]
    b2a = _tile(_rep16(a["b2"][None]), 2)
    wap = _bdiag(w_a)
    wbp = _bdiag(w_b)
    wxit = _tile(w_xi)
    bct = _tile(bc)
    w2p = jnp.concatenate([_bdiag(0.5 * w2l), _bdiag(0.5 * w2a)], axis=1)
    b2p = jnp.concatenate([_tile(0.5 * b2l), _tile(0.5 * b2a)], axis=1)
    return wap, wbp, wxit, bct, _bdiag(w_xj), w2p, b2p


def _pack_l12(p):
    l, a = p["logic"], p["alpha"]
    wc = jnp.concatenate([l["W1"], a["W1"]], axis=1)
    bc = jnp.concatenate([l["b1"], a["b1"]])[None, :]
    w_ui, w_t, w_uj, w_tp = wc[0:16], wc[16:32], wc[32:48], wc[48:64]
    wq = 0.25 * jnp.concatenate([w_t, w_tp], axis=0)
    wqp = _bdiag(wq)
    wuit = _tile(w_ui)
    bdt = _tile(bc)
    w2l1 = jnp.zeros((64, 16), jnp.float32).at[0:32].set(l["W2"])
    w2a1 = jnp.zeros((64, 16), jnp.float32).at[32:64].set(_rep16(a["W2"]))
    w2p = jnp.concatenate([_bdiag(0.5 * w2l1), _bdiag(0.5 * w2a1)], axis=1)
    b2p = jnp.concatenate([_tile(0.5 * l["b2"][None]),
                           _tile(0.5 * _rep16(a["b2"][None]))], axis=1)
    return wqp, wuit, bdt, _bdiag(w_uj), w2p, b2p


def _dot(x, w):
    return jnp.dot(x, w, preferred_element_type=jnp.float32)


def _sig(x):
    return 0.5 * jnp.tanh(0.5 * x) + 0.5


def _mlp2(x, wc, bc, w2, b2):
    h = jnp.maximum(_dot(x, wc) + bc, 0.0)
    g = _dot(h, w2) + b2
    return _sig(g[:, 0:16]) * _sig(g[:, 16:17])


def _kernel_a(x2_ref, x1_ref, wc0, bc0, w20, b20, wc1, bc1, w21, b21,
              out00_ref, out10_ref, red_ref, *, nt):
    b = pl.program_id(0)
    i = pl.program_id(1)
    j = pl.program_id(2)
    t = out10_ref.shape[1]
    blk = x2_ref[0]
    gb = blk.shape[1]
    ii = jax.lax.broadcasted_iota(jnp.int32, (t, gb, 128), 0) + i * t
    jb = jax.lax.broadcasted_iota(jnp.int32, (t, gb, 128), 1)
    ln = jax.lax.broadcasted_iota(jnp.int32, (t, gb, 128), 2)
    jl = j * t + jb * _G + ln // 16
    eq = ii == jl
    ex3 = jnp.where(eq, 0.0, blk)
    fa3 = jnp.where(eq, 1.0, blk)
    k = gb
    while k > 1:
        h = k // 2
        ex3 = jnp.maximum(ex3[:, :h], ex3[:, h:])
        fa3 = jnp.minimum(fa3[:, :h], fa3[:, h:])
        k = h
    ex2 = ex3[:, 0]
    fa2 = fa3[:, 0]
    w_ = 128
    while w_ > 16:
        h = w_ // 2
        ex2 = jnp.maximum(ex2[:, :h], ex2[:, h:])
        fa2 = jnp.minimum(fa2[:, :h], fa2[:, h:])
        w_ = h
    ex = ex2
    fa = fa2
    prev = red_ref[...]
    ex = jnp.where(j == 0, ex, jnp.maximum(prev[:, 0:16], ex))
    fa = jnp.where(j == 0, fa, jnp.minimum(prev[:, 16:32], fa))
    red_ref[...] = jnp.concatenate([ex, fa], axis=-1)

    @pl.when(j == nt - 1)
    def _():
        x1i = x1_ref[0, pl.ds(i * t, t), :]
        red = red_ref[...]
        f1 = jnp.concatenate([x1i, red], axis=-1)
        out10_ref[0] = _mlp2(f1, wc1[...], bc1[...], w21[...], b21[...])

    @pl.when(jnp.logical_and(i == 0, j == 0))
    def _():
        x1f = x1_ref[0]
        r1 = jnp.concatenate([jnp.max(x1f, axis=0), jnp.min(x1f, axis=0)])[None, :]
        out00_ref[pl.ds(b, 1), :] = _mlp2(r1, wc0[...], bc0[...], w20[...], b20[...])


def _kernel_m(x2a_ref, x2b_ref, x1i_ref, x1jp_ref, u10_ref, u10p_ref, u00_ref,
              wap, wbp, wxit, bct, wxj, w2p, b2p,
              wqp, wuit, bdt, wuj, w2p1, b2p1,
              wd1, bd1, w2d1, b2d1,
              wd0, bd0, w2d0, b2d0,
              out2_ref, out1_ref, out0_ref, red_ref, *, nti, ntj):
    b = pl.program_id(0)
    i = pl.program_id(1)
    j = pl.program_id(2)
    ti = x1i_ref.shape[1]
    gb = x1jp_ref.shape[1]
    tj = gb * _G
    m8 = ti * gb

    a2 = x2a_ref[0].reshape(m8, 128)
    bt2 = x2b_ref[0].reshape(m8, 128)
    x1i = x1i_ref[0]
    x1jp = x1jp_ref[0]

    h2 = _dot(a2, wap[...]) + _dot(bt2, wbp[...])
    hxi = _dot(x1i, wxit[...]) + bct[...]
    hxj = _dot(x1jp, wxj[...])
    h3 = h2.reshape(ti, gb, 1024) + hxi[:, None, :] + hxj[None, :, :]
    h = jnp.maximum(h3, 0.0).reshape(m8, 1024)
    g = jnp.tanh(_dot(h, w2p[...]) + b2p[...])
    r = (1.0 + g[:, 0:256]) * (1.0 + g[:, 256:512])

    r3 = r.reshape(ti, gb, 256)
    ii = jax.lax.broadcasted_iota(jnp.int32, (ti, gb, 256), 0) + i * ti
    jb = jax.lax.broadcasted_iota(jnp.int32, (ti, gb, 256), 1)
    ln = jax.lax.broadcasted_iota(jnp.int32, (ti, gb, 256), 2)
    jj = j * tj + jb * _G + ln // 32
    eq = jnp.logical_and(ii == jj, (ln % 32) < 16)
    ex3 = jnp.where(eq, 0.0, r3)
    fa3 = jnp.where(eq, 4.0, r3)
    k = gb
    while k > 1:
        h_ = k // 2
        ex3 = jnp.maximum(ex3[:, :h_], ex3[:, h_:])
        fa3 = jnp.minimum(fa3[:, :h_], fa3[:, h_:])
        k = h_
    ex2 = ex3[:, 0]
    fa2 = fa3[:, 0]
    w_ = 256
    while w_ > 32:
        h_ = w_ // 2
        ex2 = jnp.maximum(ex2[:, :h_], ex2[:, h_:])
        fa2 = jnp.minimum(fa2[:, :h_], fa2[:, h_:])
        w_ = h_
    ex = ex2[:, 0:16]
    fa = fa2[:, 0:16]
    prev = red_ref[...]
    ex = jnp.where(j == 0, ex, jnp.maximum(prev[:, 0:16], ex))
    fa = jnp.where(j == 0, fa, jnp.minimum(prev[:, 16:32], fa))
    red_ref[...] = jnp.concatenate([ex, fa], axis=-1)

    u1i = u10_ref[0, pl.ds(i * ti, ti), :]
    u1jp = u10p_ref[0, pl.ds(j * gb, gb), :]
    q2 = _dot(r, wqp[...])
    qxi = _dot(u1i, wuit[...]) + bdt[...]
    qxj = _dot(u1jp, wuj[...])
    q3 = q2.reshape(ti, gb, 512) + qxi[:, None, :] + qxj[None, :, :]
    h1 = jnp.maximum(q3, 0.0).reshape(m8, 512)
    g1 = jnp.tanh(_dot(h1, w2p1[...]) + b2p1[...])
    o = (0.25 + 0.25 * g1[:, 0:128]) * (1.0 + g1[:, 128:256])
    out2_ref[0] = o.reshape(ti, gb, 128)

    @pl.when(j == ntj - 1)
    def _():
        red = red_ref[...]
        u00b = jnp.broadcast_to(u00_ref[pl.ds(b, 1), :], (ti, 16))
        f1 = jnp.concatenate([u00b, u1i, red], axis=-1)
        out1_ref[0] = _mlp2(f1, wd1[...], bd1[...], w2d1[...], b2d1[...])

    @pl.when(jnp.logical_and(j == ntj - 1, i == nti - 1))
    def _():
        u1f = u10_ref[0]
        r1 = jnp.concatenate([jnp.max(u1f, axis=0), jnp.min(u1f, axis=0)])[None, :]
        f0 = jnp.concatenate([u00_ref[pl.ds(b, 1), :], r1], axis=-1)
        out0_ref[pl.ds(b, 1), :] = _mlp2(f0, wd0[...], bd0[...], w2d0[...], b2d0[...])


@jax.jit
def kernel(x1, x2, params):
    bsz, n, c = x1.shape

    p00, p01, p02 = params[0]
    p10, p11, p12 = params[1]
    wa0 = _pack_mlp(p00)
    wa1 = _pack_mlp(p01)
    wm2 = _pack_l02(p02)
    wq2 = _pack_l12(p12)
    wd1 = _pack_mlp(p11)
    wd1 = (wd1[0].at[32:64].multiply(0.25), *wd1[1:])
    wd0 = _pack_mlp(p10)

    x2p = x2.reshape(bsz, n, n // _G, _G * c)
    x2tp = jnp.swapaxes(x2, 1, 2).reshape(bsz, n, n // _G, _G * c)
    ta = _TTI
    nta = n // ta
    wfull_t = [pl.BlockSpec(w.shape, functools.partial(lambda nd, b, i, j: (0,) * nd, w.ndim))
               for w in (*wa0, *wa1)]
    out00, out10 = pl.pallas_call(
        functools.partial(_kernel_a, nt=nta),
        grid=(bsz, nta, nta),
        in_specs=[
            pl.BlockSpec((1, ta, ta // _G, _G * c), lambda b, i, j: (b, i, j, 0)),
            pl.BlockSpec((1, n, c), lambda b, i, j: (b, 0, 0)),
            *wfull_t,
        ],
        out_specs=[
            pl.BlockSpec((bsz, c), lambda b, i, j: (0, 0)),
            pl.BlockSpec((1, ta, c), lambda b, i, j: (b, i, 0)),
        ],
        out_shape=[
            jax.ShapeDtypeStruct((bsz, c), jnp.float32),
            jax.ShapeDtypeStruct((bsz, n, c), jnp.float32),
        ],
        scratch_shapes=[pltpu.VMEM((ta, 2 * c), jnp.float32)],
    )(x2p, x1, *wa0, *wa1)

    ti, tj = _TMI, _TMJ
    nti, ntj = n // ti, n // tj
    weights_m = (*wm2, *wq2, *wd1, *wd0)
    wfull_m = [pl.BlockSpec(w.shape, functools.partial(lambda nd, b, i, j: (0,) * nd, w.ndim))
               for w in weights_m]
    gb = tj // _G
    out2p, out1, out0 = pl.pallas_call(
        functools.partial(_kernel_m, nti=nti, ntj=ntj),
        grid=(bsz, nti, ntj),
        in_specs=[
            pl.BlockSpec((1, ti, gb, _G * c), lambda b, i, j: (b, i, j, 0)),
            pl.BlockSpec((1, ti, gb, _G * c), lambda b, i, j: (b, i, j, 0)),
            pl.BlockSpec((1, ti, c), lambda b, i, j: (b, i, 0)),
            pl.BlockSpec((1, gb, _G * c), lambda b, i, j: (b, j, 0)),
            pl.BlockSpec((1, n, c), lambda b, i, j: (b, 0, 0)),
            pl.BlockSpec((1, n // _G, _G * c), lambda b, i, j: (b, 0, 0)),
            pl.BlockSpec((bsz, c), lambda b, i, j: (0, 0)),
            *wfull_m,
        ],
        out_specs=[
            pl.BlockSpec((1, ti, gb, _G * c), lambda b, i, j: (b, i, j, 0)),
            pl.BlockSpec((1, ti, c), lambda b, i, j: (b, i, 0)),
            pl.BlockSpec((bsz, c), lambda b, i, j: (0, 0)),
        ],
        out_shape=[
            jax.ShapeDtypeStruct((bsz, n, n // _G, _G * c), jnp.float32),
            jax.ShapeDtypeStruct((bsz, n, c), jnp.float32),
            jax.ShapeDtypeStruct((bsz, c), jnp.float32),
        ],
        scratch_shapes=[pltpu.VMEM((ti, 2 * c), jnp.float32)],
    )(x2p, x2tp, x1, x1.reshape(bsz, n // _G, _G * c), out10,
      out10.reshape(bsz, n // _G, _G * c), out00, *weights_m)

    return (out0, out1, out2p.reshape(bsz, n, n, c))


# speedup vs baseline: 1.9839x; 1.0248x over previous
import functools

import jax
import jax.numpy as jnp
from jax.experimental import pallas as pl
from jax.experimental.pallas import tpu as pltpu

_TTI = 128
_TTJ = 64
_TMI = 128
_TMJ = 256
_G = 8


def _bdiag(w, g=_G):
    k, nn = w.shape
    out = jnp.zeros((g * k, g * nn), jnp.float32)
    for q in range(g):
        out = out.at[q * k:(q + 1) * k, q * nn:(q + 1) * nn].set(w)
    return out


def _tile(w, g=_G):
    return jnp.concatenate([w] * g, axis=-1)


def _rep16(w):
    return jnp.broadcast_to(w, (w.shape[0], 16))


def _pack_mlp(p):
    l, a = p["logic"], p["alpha"]
    wc = jnp.concatenate([l["W1"], a["W1"]], axis=1)
    bc = jnp.concatenate([l["b1"], a["b1"]])[None, :]
    w2 = jnp.zeros((64, 17), jnp.float32)
    w2 = w2.at[0:32, 0:16].set(l["W2"]).at[32:64, 16:17].set(a["W2"])
    b2 = jnp.concatenate([l["b2"], a["b2"]])[None, :]
    return wc, bc, w2, b2


def _pack_l02(p):
    l, a = p["logic"], p["alpha"]
    perm = lambda w: jnp.concatenate([w[32:64], w[0:32]], axis=0)
    wc = jnp.concatenate([l["W1"], a["W1"], perm(l["W1"]), perm(a["W1"])], axis=1)
    bc = jnp.concatenate([l["b1"], a["b1"], l["b1"], a["b1"]])[None, :]
    w_xi, w_a, w_xj, w_b = wc[0:16], wc[16:32], wc[32:48], wc[48:64]
    w2l = jnp.zeros((128, 32), jnp.float32)
    w2l = w2l.at[0:32, 0:16].set(l["W2"]).at[64:96, 16:32].set(l["W2"])
    w2a = jnp.zeros((128, 32), jnp.float32)
    w2a = w2a.at[32:64, 0:16].set(_rep16(a["W2"]))
    w2a = w2a.at[96:128, 16:32].set(_rep16(a["W2"]))
    b2l = jnp.concatenate([l["b2"], l["b2"]])[None, :]
    b2a = _tile(_rep16(a["b2"][None]), 2)
    wap = _bdiag(w_a)
    wbp = _bdiag(w_b)
    wxit = _tile(w_xi)
    bct = _tile(bc)
    w2p = jnp.concatenate([_bdiag(0.5 * w2l), _bdiag(0.5 * w2a)], axis=1)
    b2p = jnp.concatenate([_tile(0.5 * b2l), _tile(0.5 * b2a)], axis=1)
    return wap, wbp, wxit, bct, _bdiag(w_xj), w2p, b2p


def _pack_l12(p):
    l, a = p["logic"], p["alpha"]
    wc = jnp.concatenate([l["W1"], a["W1"]], axis=1)
    bc = jnp.concatenate([l["b1"], a["b1"]])[None, :]
    w_ui, w_t, w_uj, w_tp = wc[0:16], wc[16:32], wc[32:48], wc[48:64]
    wq = 0.25 * jnp.concatenate([w_t, w_tp], axis=0)
    wqp = _bdiag(wq)
    wuit = _tile(w_ui)
    bdt = _tile(bc)
    w2l1 = jnp.zeros((64, 16), jnp.float32).at[0:32].set(l["W2"])
    w2a1 = jnp.zeros((64, 16), jnp.float32).at[32:64].set(_rep16(a["W2"]))
    w2p = jnp.concatenate([_bdiag(0.5 * w2l1), _bdiag(0.5 * w2a1)], axis=1)
    b2p = jnp.concatenate([_tile(0.5 * l["b2"][None]),
                           _tile(0.5 * _rep16(a["b2"][None]))], axis=1)
    return wqp, wuit, bdt, _bdiag(w_uj), w2p, b2p


def _dot(x, w):
    return jnp.dot(x, w, preferred_element_type=jnp.float32)


def _sig(x):
    return 0.5 * jnp.tanh(0.5 * x) + 0.5


def _mlp2(x, wc, bc, w2, b2):
    h = jnp.maximum(_dot(x, wc) + bc, 0.0)
    g = _dot(h, w2) + b2
    return _sig(g[:, 0:16]) * _sig(g[:, 16:17])


def _kernel_a(x2_ref, x1_ref, wc0, bc0, w20, b20, wc1, bc1, w21, b21,
              out00_ref, out10_ref, red_ref, *, nt):
    b = pl.program_id(0)
    i = pl.program_id(1)
    j = pl.program_id(2)
    t = out10_ref.shape[1]
    blk = x2_ref[0]
    gb = blk.shape[1]
    ii = jax.lax.broadcasted_iota(jnp.int32, (t, gb, 128), 0) + i * t
    jb = jax.lax.broadcasted_iota(jnp.int32, (t, gb, 128), 1)
    ln = jax.lax.broadcasted_iota(jnp.int32, (t, gb, 128), 2)
    jl = j * t + jb * _G + ln // 16
    eq = ii == jl
    ex3 = jnp.where(eq, 0.0, blk)
    fa3 = jnp.where(eq, 1.0, blk)
    k = gb
    while k > 1:
        h = k // 2
        ex3 = jnp.maximum(ex3[:, :h], ex3[:, h:])
        fa3 = jnp.minimum(fa3[:, :h], fa3[:, h:])
        k = h
    ex2 = ex3[:, 0]
    fa2 = fa3[:, 0]
    w_ = 128
    while w_ > 16:
        h = w_ // 2
        ex2 = jnp.maximum(ex2[:, :h], ex2[:, h:])
        fa2 = jnp.minimum(fa2[:, :h], fa2[:, h:])
        w_ = h
    ex = ex2
    fa = fa2
    prev = red_ref[...]
    ex = jnp.where(j == 0, ex, jnp.maximum(prev[:, 0:16], ex))
    fa = jnp.where(j == 0, fa, jnp.minimum(prev[:, 16:32], fa))
    red_ref[...] = jnp.concatenate([ex, fa], axis=-1)

    @pl.when(j == nt - 1)
    def _():
        x1i = x1_ref[0, pl.ds(i * t, t), :]
        red = red_ref[...]
        f1 = jnp.concatenate([x1i, red], axis=-1)
        out10_ref[0] = _mlp2(f1, wc1[...], bc1[...], w21[...], b21[...])

    @pl.when(jnp.logical_and(i == 0, j == 0))
    def _():
        x1f = x1_ref[0]
        r1 = jnp.concatenate([jnp.max(x1f, axis=0), jnp.min(x1f, axis=0)])[None, :]
        out00_ref[pl.ds(b, 1), :] = _mlp2(r1, wc0[...], bc0[...], w20[...], b20[...])


def _kernel_m(x2a_ref, x2b_ref, x1i_ref, x1jp_ref, u10_ref, u10p_ref, u00_ref,
              wap, wbp, wxit, bct, wxj, w2p, b2p,
              wqp, wuit, bdt, wuj, w2p1, b2p1,
              wd1, bd1, w2d1, b2d1,
              wd0, bd0, w2d0, b2d0,
              out2_ref, out1_ref, out0_ref, red_ref, *, nti, ntj):
    b = pl.program_id(0)
    i = pl.program_id(1)
    j = pl.program_id(2)
    ti = x1i_ref.shape[1]
    gb = x1jp_ref.shape[1]
    tj = gb * _G
    m8 = ti * gb

    a2 = x2a_ref[0].reshape(m8, 128)
    bt2 = x2b_ref[0].reshape(m8, 128)
    x1i = x1i_ref[0]
    x1jp = x1jp_ref[0]

    h2 = _dot(a2, wap[...]) + _dot(bt2, wbp[...])
    hxi = _dot(x1i, wxit[...]) + bct[...]
    hxj = _dot(x1jp, wxj[...])
    h3 = h2.reshape(ti, gb, 1024) + hxi[:, None, :] + hxj[None, :, :]
    h = jnp.maximum(h3, 0.0).reshape(m8, 1024)
    g = jnp.tanh(_dot(h, w2p[...]) + b2p[...])
    r = (1.0 + g[:, 0:256]) * (1.0 + g[:, 256:512])

    r3 = r.reshape(ti, gb, 256)
    ii = jax.lax.broadcasted_iota(jnp.int32, (ti, gb, 256), 0) + i * ti
    jb = jax.lax.broadcasted_iota(jnp.int32, (ti, gb, 256), 1)
    ln = jax.lax.broadcasted_iota(jnp.int32, (ti, gb, 256), 2)
    jj = j * tj + jb * _G + ln // 32
    eq = jnp.logical_and(ii == jj, (ln % 32) < 16)
    ex3 = jnp.where(eq, 0.0, r3)
    fa3 = jnp.where(eq, 4.0, r3)
    k = gb
    while k > 1:
        h_ = k // 2
        ex3 = jnp.maximum(ex3[:, :h_], ex3[:, h_:])
        fa3 = jnp.minimum(fa3[:, :h_], fa3[:, h_:])
        k = h_
    ex2 = ex3[:, 0]
    fa2 = fa3[:, 0]
    w_ = 256
    while w_ > 32:
        h_ = w_ // 2
        ex2 = jnp.maximum(ex2[:, :h_], ex2[:, h_:])
        fa2 = jnp.minimum(fa2[:, :h_], fa2[:, h_:])
        w_ = h_
    ex = ex2[:, 0:16]
    fa = fa2[:, 0:16]
    prev = red_ref[...]
    ex = jnp.where(j == 0, ex, jnp.maximum(prev[:, 0:16], ex))
    fa = jnp.where(j == 0, fa, jnp.minimum(prev[:, 16:32], fa))
    red_ref[...] = jnp.concatenate([ex, fa], axis=-1)

    u1i = u10_ref[0, pl.ds(i * ti, ti), :]
    u1jp = u10p_ref[0, pl.ds(j * gb, gb), :]
    q2 = _dot(r, wqp[...])
    qxi = _dot(u1i, wuit[...]) + bdt[...]
    qxj = _dot(u1jp, wuj[...])
    q3 = q2.reshape(ti, gb, 512) + qxi[:, None, :] + qxj[None, :, :]
    h1 = jnp.maximum(q3, 0.0).reshape(m8, 512)
    g1 = jnp.tanh(_dot(h1, w2p1[...]) + b2p1[...])
    o = (0.25 + 0.25 * g1[:, 0:128]) * (1.0 + g1[:, 128:256])
    out2_ref[0] = o.reshape(ti, gb, 128)

    @pl.when(j == ntj - 1)
    def _():
        red = red_ref[...]
        u00b = jnp.broadcast_to(u00_ref[pl.ds(b, 1), :], (ti, 16))
        f1 = jnp.concatenate([u00b, u1i, red], axis=-1)
        out1_ref[0] = _mlp2(f1, wd1[...], bd1[...], w2d1[...], b2d1[...])

    @pl.when(jnp.logical_and(j == ntj - 1, i == nti - 1))
    def _():
        u1f = u10_ref[0]
        r1 = jnp.concatenate([jnp.max(u1f, axis=0), jnp.min(u1f, axis=0)])[None, :]
        f0 = jnp.concatenate([u00_ref[pl.ds(b, 1), :], r1], axis=-1)
        out0_ref[pl.ds(b, 1), :] = _mlp2(f0, wd0[...], bd0[...], w2d0[...], b2d0[...])


@jax.jit
def kernel(x1, x2, params):
    bsz, n, c = x1.shape

    p00, p01, p02 = params[0]
    p10, p11, p12 = params[1]
    wa0 = _pack_mlp(p00)
    wa1 = _pack_mlp(p01)
    wm2 = _pack_l02(p02)
    wq2 = _pack_l12(p12)
    wd1 = _pack_mlp(p11)
    wd1 = (wd1[0].at[32:64].multiply(0.25), *wd1[1:])
    wd0 = _pack_mlp(p10)

    x2p = x2.reshape(bsz, n, n // _G, _G * c)
    x2tp = jnp.swapaxes(x2, 1, 2).reshape(bsz, n, n // _G, _G * c)
    ta = _TTI
    nta = n // ta
    wfull_t = [pl.BlockSpec(w.shape, functools.partial(lambda nd, b, i, j: (0,) * nd, w.ndim))
               for w in (*wa0, *wa1)]
    out00, out10 = pl.pallas_call(
        functools.partial(_kernel_a, nt=nta),
        grid=(bsz, nta, nta),
        in_specs=[
            pl.BlockSpec((1, ta, ta // _G, _G * c), lambda b, i, j: (b, i, j, 0)),
            pl.BlockSpec((1, n, c), lambda b, i, j: (b, 0, 0)),
            *wfull_t,
        ],
        out_specs=[
            pl.BlockSpec((bsz, c), lambda b, i, j: (0, 0)),
            pl.BlockSpec((1, ta, c), lambda b, i, j: (b, i, 0)),
        ],
        out_shape=[
            jax.ShapeDtypeStruct((bsz, c), jnp.float32),
            jax.ShapeDtypeStruct((bsz, n, c), jnp.float32),
        ],
        scratch_shapes=[pltpu.VMEM((ta, 2 * c), jnp.float32)],
    )(x2p, x1, *wa0, *wa1)

    ti, tj = _TMI, _TMJ
    nti, ntj = n // ti, n // tj
    weights_m = (*wm2, *wq2, *wd1, *wd0)
    wfull_m = [pl.BlockSpec(w.shape, functools.partial(lambda nd, b, i, j: (0,) * nd, w.ndim))
               for w in weights_m]
    gb = tj // _G
    out2p, out1, out0 = pl.pallas_call(
        functools.partial(_kernel_m, nti=nti, ntj=ntj),
        grid=(bsz, nti, ntj),
        in_specs=[
            pl.BlockSpec((1, ti, gb, _G * c), lambda b, i, j: (b, i, j, 0)),
            pl.BlockSpec((1, ti, gb, _G * c), lambda b, i, j: (b, i, j, 0)),
            pl.BlockSpec((1, ti, c), lambda b, i, j: (b, i, 0)),
            pl.BlockSpec((1, gb, _G * c), lambda b, i, j: (b, j, 0)),
            pl.BlockSpec((1, n, c), lambda b, i, j: (b, 0, 0)),
            pl.BlockSpec((1, n // _G, _G * c), lambda b, i, j: (b, 0, 0)),
            pl.BlockSpec((bsz, c), lambda b, i, j: (0, 0)),
            *wfull_m,
        ],
        out_specs=[
            pl.BlockSpec((1, ti, gb, _G * c), lambda b, i, j: (b, i, j, 0)),
            pl.BlockSpec((1, ti, c), lambda b, i, j: (b, i, 0)),
            pl.BlockSpec((bsz, c), lambda b, i, j: (0, 0)),
        ],
        out_shape=[
            jax.ShapeDtypeStruct((bsz, n, n // _G, _G * c), jnp.float32),
            jax.ShapeDtypeStruct((bsz, n, c), jnp.float32),
            jax.ShapeDtypeStruct((bsz, c), jnp.float32),
        ],
        scratch_shapes=[pltpu.VMEM((ti, 2 * c), jnp.float32)],
    )(x2p, x2tp, x1, x1.reshape(bsz, n // _G, _G * c), out10,
      out10.reshape(bsz, n // _G, _G * c), out00, *weights_m)

    return (out0, out1, out2p.reshape(bsz, n, n, c))
